# baseline (speedup 1.0000x reference)
import jax
import jax.numpy as jnp
from jax.experimental import pallas as pl
from jax.experimental.pallas import tpu as pltpu

N_NODES = 50
N_EDGES = 800
EMB = 5
F32 = jnp.float32


def _lrelu(x):
    return jnp.where(x >= 0, x, 0.01 * x)


def _sigmoid(x):
    return 1.0 / (1.0 + jnp.exp(-x))


def _dot(a, b):
    return jax.lax.dot_general(a, b, (((1,), (0,)), ((), ())),
                               preferred_element_type=F32)


def _fused(ei_ref, data_ref, dflat_ref, g5_ref,
           fW1p_ref, g128_ref, g64_ref, g2_ref, out_ref):
    src = ei_ref[0:1, :]
    dst = ei_ref[1:2, :]
    rows = jax.lax.broadcasted_iota(jnp.int32, (N_NODES, N_EDGES), 0)
    m_dst = (rows == dst).astype(F32)
    m_src = (rows == src).astype(F32)
    A = jax.lax.dot_general(m_dst, m_src, (((1,), (1,)), ((), ())),
                            preferred_element_type=F32)

    x0 = data_ref[...]
    ax0 = _dot(A, x0)
    t0 = ax0 * g5_ref[0:1, :]
    h = _lrelu(x0 * g5_ref[1:2, :] + t0)
    g = _sigmoid(x0 * g5_ref[2:3, :] + t0)
    x = jnp.concatenate([h, g * h], axis=1)

    for base in (3, 33):
        W = g5_ref[base:base + 10, :]
        U = g5_ref[base + 10:base + 20, :]
        G = g5_ref[base + 20:base + 30, :]
        ax = _dot(A, x)
        t = _dot(ax, W)
        h = _lrelu(_dot(x, U) + t)
        g = _sigmoid(_dot(x, G) + t)
        x = jnp.concatenate([h, g * h], axis=1)

    xt = jnp.transpose(x)
    vecx = jnp.concatenate([xt[j:j + 1, :] for j in range(2 * EMB)], axis=1)
    full = jnp.concatenate([vecx, dflat_ref[...]], axis=1)

    fW2 = g128_ref[0:128, :]
    fb1 = g128_ref[128:129, :]
    fb2 = g128_ref[129:130, :]
    fW3 = g64_ref[0:128, :]
    fb3 = g64_ref[128:129, :]
    fW4 = g2_ref[0:64, :]
    fb4 = g2_ref[64:65, :]
    h1 = _lrelu(_dot(full, fW1p_ref[...]) + fb1)
    h2 = _lrelu(_dot(h1, fW2) + fb2)
    h3 = _lrelu(_dot(h2, fW3) + fb3)
    y = _sigmoid(_dot(h3, fW4) + fb4)
    out_ref[...] = 0.5 * (y[0, 0] + y[0, 1])


def kernel(data, d, edge_index, W0, U0, G0, W1, U1, G1, W2, U2, G2,
           fW1, fb1, fW2, fb2, fW3, fb3, fW4, fb4):
    dflat = d.reshape(1, -1)
    g5 = jnp.concatenate(
        [W0, U0, G0, W1, U1, G1, W2, U2, G2], axis=0)
    g128 = jnp.concatenate(
        [fW2, fb1.reshape(1, -1), fb2.reshape(1, -1)], axis=0)
    g64 = jnp.concatenate([fW3, fb3.reshape(1, -1)], axis=0)
    g2 = jnp.concatenate([fW4, fb4.reshape(1, -1)], axis=0)
    fW1x = fW1[:N_NODES * 2 * EMB].reshape(N_NODES, 2 * EMB, -1)
    fW1p = jnp.concatenate(
        [fW1x.transpose(1, 0, 2).reshape(N_NODES * 2 * EMB, -1),
         fW1[N_NODES * 2 * EMB:]], axis=0)
    out = pl.pallas_call(
        _fused,
        out_shape=jax.ShapeDtypeStruct((), F32),
        out_specs=pl.BlockSpec(memory_space=pltpu.SMEM),
        compiler_params=pltpu.CompilerParams(
            allow_input_fusion=(True,) * 8),
    )(edge_index, data, dflat, g5, fW1p, g128, g64, g2)
    return out


# speedup vs baseline: 1.0355x; 1.0355x over previous
import jax
import jax.numpy as jnp
from jax.experimental import pallas as pl
from jax.experimental.pallas import tpu as pltpu

N_NODES = 50
N_EDGES = 800
EMB = 5
F32 = jnp.float32


def _lrelu(x):
    return jnp.where(x >= 0, x, 0.01 * x)


def _sigmoid(x):
    return 1.0 / (1.0 + jnp.exp(-x))


def _dot(a, b):
    return jax.lax.dot_general(a, b, (((1,), (0,)), ((), ())),
                               preferred_element_type=F32)


def _fused(ei_ref, data_ref, dflat_ref, g5_ref,
           fW1_ref, g128_ref, g64_ref, g2_ref, out_ref):
    src = ei_ref[0:1, :]
    dst = ei_ref[1:2, :]
    rows = jax.lax.broadcasted_iota(jnp.int32, (N_NODES, N_EDGES), 0)
    m_dst = (rows == dst).astype(F32)
    m_src = (rows == src).astype(F32)
    A = jax.lax.dot_general(m_dst, m_src, (((1,), (1,)), ((), ())),
                            preferred_element_type=F32)

    x0 = data_ref[...]
    ax0 = _dot(A, x0)
    t0 = ax0 * g5_ref[0:1, :]
    h = _lrelu(x0 * g5_ref[1:2, :] + t0)
    g = _sigmoid(x0 * g5_ref[2:3, :] + t0)
    x = jnp.concatenate([h, g * h], axis=1)

    for base in (3, 33):
        W = g5_ref[base:base + 10, :]
        U = g5_ref[base + 10:base + 20, :]
        G = g5_ref[base + 20:base + 30, :]
        ax = _dot(A, x)
        t = _dot(ax, W)
        h = _lrelu(_dot(x, U) + t)
        g = _sigmoid(_dot(x, G) + t)
        x = jnp.concatenate([h, g * h], axis=1)

    xt = jnp.transpose(x)
    vecx = jnp.concatenate([xt[j:j + 1, :] for j in range(2 * EMB)], axis=1)
    nvec = N_NODES * 2 * EMB
    pr = jax.lax.broadcasted_iota(jnp.int32, (nvec, nvec), 0)
    pc = jax.lax.broadcasted_iota(jnp.int32, (nvec, nvec), 1)
    perm = (pc == (pr % N_NODES) * (2 * EMB) + pr // N_NODES).astype(F32)
    vecx = _dot(vecx, perm)
    full = jnp.concatenate([vecx, dflat_ref[...]], axis=1)

    fW2 = g128_ref[0:128, :]
    fb1 = g128_ref[128:129, :]
    fb2 = g128_ref[129:130, :]
    fW3 = g64_ref[0:128, :]
    fb3 = g64_ref[128:129, :]
    fW4 = g2_ref[0:64, :]
    fb4 = g2_ref[64:65, :]
    h1 = _lrelu(_dot(full, fW1_ref[...]) + fb1)
    h2 = _lrelu(_dot(h1, fW2) + fb2)
    h3 = _lrelu(_dot(h2, fW3) + fb3)
    y = _sigmoid(_dot(h3, fW4) + fb4)
    out_ref[...] = 0.5 * (y[0, 0] + y[0, 1])


def kernel(data, d, edge_index, W0, U0, G0, W1, U1, G1, W2, U2, G2,
           fW1, fb1, fW2, fb2, fW3, fb3, fW4, fb4):
    dflat = d.reshape(1, -1)
    g5 = jnp.concatenate(
        [W0, U0, G0, W1, U1, G1, W2, U2, G2], axis=0)
    g128 = jnp.concatenate(
        [fW2, fb1.reshape(1, -1), fb2.reshape(1, -1)], axis=0)
    g64 = jnp.concatenate([fW3, fb3.reshape(1, -1)], axis=0)
    g2 = jnp.concatenate([fW4, fb4.reshape(1, -1)], axis=0)
    out = pl.pallas_call(
        _fused,
        out_shape=jax.ShapeDtypeStruct((), F32),
        out_specs=pl.BlockSpec(memory_space=pltpu.SMEM),
        compiler_params=pltpu.CompilerParams(
            allow_input_fusion=(True,) * 8),
    )(edge_index, data, dflat, g5, fW1, g128, g64, g2)
    return out
